# baseline (speedup 1.0000x reference)
import functools

import numpy as np
import jax
import jax.numpy as jnp
from jax import lax
from jax.experimental import pallas as pl
from jax.experimental.pallas import tpu as pltpu
from jax.experimental.pallas import tpu_sc as plsc

D_MODEL = 768
POS_LENGTH = 2048
SCALE = float(np.sqrt(float(D_MODEL)))

NUM_CORES = 2
NUM_SUBCORES = 16
NUM_WORKERS = NUM_CORES * NUM_SUBCORES
LANES = 16

ROWS_TOTAL = 8192
ROWS_PER_W = ROWS_TOTAL // NUM_WORKERS
CHUNK = 64
N_CHUNKS = ROWS_PER_W // CHUNK
VECS_PER_ROW = D_MODEL // LANES


def _positional_encoding(length: int, depth: int) -> np.ndarray:
    depth_half = depth / 2
    positions = np.arange(length)[:, np.newaxis].astype(np.float32)
    depths = (np.arange(depth_half)[np.newaxis, :] / depth_half).astype(np.float32)
    angle_rates = 1.0 / (10000.0 ** depths)
    angle_rads = positions * angle_rates
    return np.concatenate([np.sin(angle_rads), np.cos(angle_rads)], axis=-1)


_POS_NP = _positional_encoding(POS_LENGTH, D_MODEL)


@functools.partial(
    pl.kernel,
    out_type=jax.ShapeDtypeStruct((ROWS_TOTAL, D_MODEL), jnp.float32),
    mesh=plsc.VectorSubcoreMesh(core_axis_name="c", subcore_axis_name="s"),
    scratch_types=[
        pltpu.VMEM((ROWS_PER_W,), jnp.int32),
        pltpu.VMEM((CHUNK, D_MODEL), jnp.float32),
        pltpu.VMEM((CHUNK, D_MODEL), jnp.float32),
        pltpu.SemaphoreType.DMA,
    ],
)
def _emb_kernel(table_hbm, idx_hbm, pos_hbm, out_hbm, idx_v, rows_v, pos_v, sem):
    wid = lax.axis_index("s") * NUM_CORES + lax.axis_index("c")
    base = wid * ROWS_PER_W
    l_base = lax.rem(base, POS_LENGTH)

    pltpu.sync_copy(idx_hbm.at[pl.ds(base, ROWS_PER_W)], idx_v)

    for c in range(N_CHUNKS):
        row0 = base + c * CHUNK
        l0 = l_base + c * CHUNK
        pltpu.async_copy(
            table_hbm.at[idx_v.at[pl.ds(c * CHUNK, CHUNK)]], rows_v, sem
        ).wait()
        pltpu.sync_copy(pos_hbm.at[pl.ds(l0, CHUNK)], pos_v)

        def row_body(r, _):
            def vec_body(j, _):
                sl = pl.ds(j * LANES, LANES)
                rows_v[r, sl] = rows_v[r, sl] * SCALE + pos_v[r, sl]
                return 0

            lax.fori_loop(0, VECS_PER_ROW, vec_body, 0)
            return 0

        lax.fori_loop(0, CHUNK, row_body, 0)

        pltpu.sync_copy(rows_v, out_hbm.at[pl.ds(row0, CHUNK)])


def kernel(x, table):
    b, l = x.shape
    idx = x.reshape(b * l).astype(jnp.int32)
    pos = jnp.asarray(_POS_NP, dtype=jnp.float32)
    out = _emb_kernel(table, idx, pos)
    return out.reshape(b, l, D_MODEL)


# speedup vs baseline: 2.1929x; 2.1929x over previous
import functools

import numpy as np
import jax
import jax.numpy as jnp
from jax import lax
from jax.experimental import pallas as pl
from jax.experimental.pallas import tpu as pltpu
from jax.experimental.pallas import tpu_sc as plsc

D_MODEL = 768
POS_LENGTH = 2048
SCALE = float(np.sqrt(float(D_MODEL)))

NUM_CORES = 2
NUM_SUBCORES = 16
NUM_WORKERS = NUM_CORES * NUM_SUBCORES
LANES = 16

ROWS_TOTAL = 8192
ROWS_PER_W = ROWS_TOTAL // NUM_WORKERS
CHUNK = 32
N_CHUNKS = ROWS_PER_W // CHUNK
VECS_PER_ROW = D_MODEL // LANES
N_RBUF = 3
N_PBUF = 2


def _positional_encoding(length: int, depth: int) -> np.ndarray:
    depth_half = depth / 2
    positions = np.arange(length)[:, np.newaxis].astype(np.float32)
    depths = (np.arange(depth_half)[np.newaxis, :] / depth_half).astype(np.float32)
    angle_rates = 1.0 / (10000.0 ** depths)
    angle_rads = positions * angle_rates
    return np.concatenate([np.sin(angle_rads), np.cos(angle_rads)], axis=-1)


_POS_NP = _positional_encoding(POS_LENGTH, D_MODEL)


@functools.partial(
    pl.kernel,
    out_type=jax.ShapeDtypeStruct((ROWS_TOTAL, D_MODEL), jnp.float32),
    mesh=plsc.VectorSubcoreMesh(core_axis_name="c", subcore_axis_name="s"),
    scratch_types=(
        [pltpu.VMEM((ROWS_PER_W,), jnp.int32)]
        + [pltpu.VMEM((CHUNK, D_MODEL), jnp.float32)] * (N_RBUF + N_PBUF)
        + [pltpu.SemaphoreType.DMA] * (2 * N_RBUF + N_PBUF)
    ),
)
def _emb_kernel(table_hbm, idx_hbm, pos_hbm, out_hbm, idx_v,
                r0, r1, r2, p0, p1,
                gs0, gs1, gs2, os0, os1, os2, ps0, ps1):
    rows = [r0, r1, r2]
    posb = [p0, p1]
    gsem = [gs0, gs1, gs2]
    osem = [os0, os1, os2]
    psem = [ps0, ps1]

    wid = lax.axis_index("s") * NUM_CORES + lax.axis_index("c")
    base = wid * ROWS_PER_W
    l_base = lax.rem(base, POS_LENGTH)

    pltpu.sync_copy(idx_hbm.at[pl.ds(base, ROWS_PER_W)], idx_v)

    def start_in(c):
        rb, pb = c % N_RBUF, c % N_PBUF
        g = pltpu.async_copy(
            table_hbm.at[idx_v.at[pl.ds(c * CHUNK, CHUNK)]], rows[rb], gsem[rb]
        )
        p = pltpu.async_copy(
            pos_hbm.at[pl.ds(l_base + c * CHUNK, CHUNK)], posb[pb], psem[pb]
        )
        return g, p

    in_fl = {0: start_in(0)}
    out_fl = {}
    for c in range(N_CHUNKS):
        rb, pb = c % N_RBUF, c % N_PBUF
        g, p = in_fl.pop(c)
        g.wait()
        p.wait()
        if c + 1 - N_RBUF in out_fl:
            out_fl.pop(c + 1 - N_RBUF).wait()
        if c + 1 < N_CHUNKS:
            in_fl[c + 1] = start_in(c + 1)

        r_ref, p_ref = rows[rb], posb[pb]

        @plsc.parallel_loop(0, CHUNK, 1, unroll=2)
        def row_body(r):
            for j in range(VECS_PER_ROW):
                sl = pl.ds(j * LANES, LANES)
                r_ref[r, sl] = r_ref[r, sl] * SCALE + p_ref[r, sl]

        out_fl[c] = pltpu.async_copy(
            rows[rb], out_hbm.at[pl.ds(base + c * CHUNK, CHUNK)], osem[rb]
        )

    for c in sorted(out_fl):
        out_fl.pop(c).wait()


def kernel(x, table):
    b, l = x.shape
    idx = x.reshape(b * l).astype(jnp.int32)
    pos = jnp.asarray(_POS_NP, dtype=jnp.float32)
    out = _emb_kernel(table, idx, pos)
    return out.reshape(b, l, D_MODEL)
